# Initial kernel scaffold; baseline (speedup 1.0000x reference)
import jax
import jax.numpy as jnp
from jax.experimental import pallas as pl


def kernel(Wx, edge_index):
    raise NotImplementedError("write your pallas kernel here")


# speedup vs baseline: 6.1342x; 6.1342x over previous
import functools

import jax
import jax.numpy as jnp
from jax import lax
from jax.experimental import pallas as pl
from jax.experimental.pallas import tpu as pltpu
from jax.experimental.pallas import tpu_sc as plsc

N_NODES = 10000
N_EDGES = 320000
D = 128
NC, NS = 2, 16
NW = NC * NS
EPW = N_EDGES // NW
CH = 128
NFULL = EPW // CH
TAIL = EPW - NFULL * CH

_mesh = plsc.VectorSubcoreMesh(core_axis_name="c", subcore_axis_name="s")


def _one_gather(wx, idx_v, out, base, rbuf, tbuf, gsem, wsem):
    for b in range(2):
        pltpu.async_copy(wx.at[idx_v.at[pl.ds(b * CH, CH)]], rbuf.at[b], gsem)

    def outer(i, carry):
        c0 = i * 2
        for b in range(2):
            c = c0 + b
            pltpu.make_async_copy(
                wx.at[idx_v.at[pl.ds(0, CH)]], rbuf.at[b], gsem).wait()
            pltpu.async_copy(
                rbuf.at[b], out.at[pl.ds(base + c * CH, CH)], wsem)
            pltpu.make_async_copy(
                rbuf.at[b], out.at[pl.ds(base, CH)], wsem).wait()
            pltpu.async_copy(
                wx.at[idx_v.at[pl.ds((c + 2) * CH, CH)]], rbuf.at[b], gsem)
        return carry

    lax.fori_loop(0, NFULL // 2 - 1, outer, 0)

    for b in range(2):
        c = NFULL - 2 + b
        pltpu.make_async_copy(
            wx.at[idx_v.at[pl.ds(0, CH)]], rbuf.at[b], gsem).wait()
        pltpu.async_copy(rbuf.at[b], out.at[pl.ds(base + c * CH, CH)], wsem)
    for b in range(2):
        pltpu.make_async_copy(
            rbuf.at[b], out.at[pl.ds(base, CH)], wsem).wait()

    pltpu.async_copy(
        wx.at[idx_v.at[pl.ds(NFULL * CH, TAIL)]], tbuf, gsem).wait()
    pltpu.sync_copy(tbuf, out.at[pl.ds(base + NFULL * CH, TAIL)])


@functools.partial(
    pl.kernel,
    mesh=_mesh,
    out_type=(jax.ShapeDtypeStruct((N_EDGES, D), jnp.float32),
              jax.ShapeDtypeStruct((N_EDGES, D), jnp.float32)),
    scratch_types=[
        pltpu.VMEM((EPW,), jnp.int32),
        pltpu.VMEM((EPW,), jnp.int32),
        pltpu.VMEM((2, CH, D), jnp.float32),
        pltpu.VMEM((TAIL, D), jnp.float32),
        pltpu.SemaphoreType.DMA,
        pltpu.SemaphoreType.DMA,
    ],
)
def _gather_both(wx, esrc, edst, out_i, out_j, idx_i, idx_j, rbuf, tbuf,
                 gsem, wsem):
    wid = lax.axis_index("s") * NC + lax.axis_index("c")
    base = wid * EPW
    pltpu.sync_copy(edst.at[pl.ds(base, EPW)], idx_i)
    pltpu.sync_copy(esrc.at[pl.ds(base, EPW)], idx_j)
    _one_gather(wx, idx_i, out_i, base, rbuf, tbuf, gsem, wsem)
    _one_gather(wx, idx_j, out_j, base, rbuf, tbuf, gsem, wsem)


def kernel(Wx, edge_index):
    eidx = edge_index.astype(jnp.int32)
    return _gather_both(Wx, eidx[0], eidx[1])


# speedup vs baseline: 10.1207x; 1.6499x over previous
import functools

import jax
import jax.numpy as jnp
from jax import lax
from jax.experimental import pallas as pl
from jax.experimental.pallas import tpu as pltpu
from jax.experimental.pallas import tpu_sc as plsc

N_NODES = 10000
N_EDGES = 320000
D = 128
NC, NS = 2, 16
NW = NC * NS
EPW = N_EDGES // NW
CH = 128
NFULL = EPW // CH
TAIL = EPW - NFULL * CH

_mesh = plsc.VectorSubcoreMesh(core_axis_name="c", subcore_axis_name="s")


def _one_gather(wx, idx_v, out, base, rbuf, tbuf, gsem, wsem):
    for b in range(2):
        pltpu.async_copy(wx.at[idx_v.at[pl.ds(b * CH, CH)]], rbuf.at[b], gsem)

    def outer(i, carry):
        c0 = i * 2
        for b in range(2):
            c = c0 + b
            pltpu.make_async_copy(
                wx.at[idx_v.at[pl.ds(0, CH)]], rbuf.at[b], gsem).wait()
            pltpu.async_copy(
                rbuf.at[b], out.at[pl.ds(base + c * CH, CH)], wsem)
            pltpu.make_async_copy(
                rbuf.at[b], out.at[pl.ds(base, CH)], wsem).wait()
            pltpu.async_copy(
                wx.at[idx_v.at[pl.ds((c + 2) * CH, CH)]], rbuf.at[b], gsem)
        return carry

    lax.fori_loop(0, NFULL // 2 - 1, outer, 0)

    for b in range(2):
        c = NFULL - 2 + b
        pltpu.make_async_copy(
            wx.at[idx_v.at[pl.ds(0, CH)]], rbuf.at[b], gsem).wait()
        pltpu.async_copy(rbuf.at[b], out.at[pl.ds(base + c * CH, CH)], wsem)
    for b in range(2):
        pltpu.make_async_copy(
            rbuf.at[b], out.at[pl.ds(base, CH)], wsem).wait()

    pltpu.async_copy(
        wx.at[idx_v.at[pl.ds(NFULL * CH, TAIL)]], tbuf, gsem).wait()
    pltpu.sync_copy(tbuf, out.at[pl.ds(base + NFULL * CH, TAIL)])


@functools.partial(
    pl.kernel,
    mesh=_mesh,
    out_type=(jax.ShapeDtypeStruct((N_EDGES, D), jnp.float32),
              jax.ShapeDtypeStruct((N_EDGES, D), jnp.float32)),
    scratch_types=[
        pltpu.VMEM((EPW,), jnp.int32),
        pltpu.VMEM((2, CH, D), jnp.float32),
        pltpu.VMEM((TAIL, D), jnp.float32),
        pltpu.VMEM_SHARED((N_NODES, D), jnp.float32),
        pltpu.SemaphoreType.DMA,
        pltpu.SemaphoreType.DMA,
    ],
)
def _gather_both(wx, esrc, edst, out_i, out_j, idx_v, rbuf, tbuf,
                 shared, gsem, wsem):
    wid = lax.axis_index("s") * NC + lax.axis_index("c")
    base = wid * EPW
    sid = lax.axis_index("s")
    rows = (N_NODES // NS) // 8 * 8
    off = sid * rows
    pltpu.sync_copy(wx.at[pl.ds(off, rows)], shared.at[pl.ds(off, rows)])

    @pl.when(sid == NS - 1)
    def _():
        rem_off = NS * rows
        pltpu.sync_copy(wx.at[pl.ds(rem_off, N_NODES - rem_off)],
                        shared.at[pl.ds(rem_off, N_NODES - rem_off)])

    pltpu.sync_copy(edst.at[pl.ds(base, EPW)], idx_v)
    plsc.subcore_barrier()
    _one_gather(shared, idx_v, out_i, base, rbuf, tbuf, gsem, wsem)
    pltpu.sync_copy(esrc.at[pl.ds(base, EPW)], idx_v)
    _one_gather(shared, idx_v, out_j, base, rbuf, tbuf, gsem, wsem)


def kernel(Wx, edge_index):
    eidx = edge_index.astype(jnp.int32)
    return _gather_both(Wx, eidx[0], eidx[1])


# speedup vs baseline: 10.3796x; 1.0256x over previous
import functools

import jax
import jax.numpy as jnp
from jax import lax
from jax.experimental import pallas as pl
from jax.experimental.pallas import tpu as pltpu
from jax.experimental.pallas import tpu_sc as plsc

N_NODES = 10000
N_EDGES = 320000
D = 128
NC, NS = 2, 16
NW = NC * NS
EPW = N_EDGES // NW
CH = 80
NFULL = EPW // CH
NB = 4

_mesh = plsc.VectorSubcoreMesh(core_axis_name="c", subcore_axis_name="s")


def _one_gather(wx, idx_v, out, base, rbuf, gsem, wsem):
    def g(c, b):
        pltpu.async_copy(wx.at[idx_v.at[pl.ds(c * CH, CH)]], rbuf.at[b], gsem)

    def gwait(b):
        pltpu.make_async_copy(
            wx.at[idx_v.at[pl.ds(0, CH)]], rbuf.at[b], gsem).wait()

    def w(c, b):
        pltpu.async_copy(rbuf.at[b], out.at[pl.ds(base + c * CH, CH)], wsem)

    def wwait():
        pltpu.make_async_copy(
            rbuf.at[0], out.at[pl.ds(base, CH)], wsem).wait()

    g(0, 0)
    g(1, 1)
    gwait(0); w(0, 0); g(2, 2)
    gwait(1); w(1, 1); g(3, 3)
    gwait(2); w(2, 2); wwait(); g(4, 0)
    gwait(3); w(3, 3); wwait(); g(5, 1)

    def outer(i, carry):
        c0 = 4 + i * 4
        for b in range(4):
            c = c0 + b
            bb = c % 4
            gwait(bb)
            w(c, bb)
            wwait()
            g(c + 2, (c + 2) % 4)
        return carry

    lax.fori_loop(0, (NFULL - 5 - 4) // 4, outer, 0)

    for c in (120, 121, 122):
        gwait(c % 4); w(c, c % 4); wwait(); g(c + 2, (c + 2) % 4)
    for c in (123, 124):
        gwait(c % 4); w(c, c % 4)
    for _ in range(4):
        wwait()


@functools.partial(
    pl.kernel,
    mesh=_mesh,
    out_type=(jax.ShapeDtypeStruct((N_EDGES, D), jnp.float32),
              jax.ShapeDtypeStruct((N_EDGES, D), jnp.float32)),
    scratch_types=[
        pltpu.VMEM((EPW,), jnp.int32),
        pltpu.VMEM((NB, CH, D), jnp.float32),
        pltpu.VMEM_SHARED((N_NODES, D), jnp.float32),
        pltpu.SemaphoreType.DMA,
        pltpu.SemaphoreType.DMA,
    ],
)
def _gather_both(wx, esrc, edst, out_i, out_j, idx_v, rbuf, shared,
                 gsem, wsem):
    wid = lax.axis_index("s") * NC + lax.axis_index("c")
    base = wid * EPW
    sid = lax.axis_index("s")
    rows = (N_NODES // NS) // 8 * 8
    off = sid * rows
    pltpu.sync_copy(wx.at[pl.ds(off, rows)], shared.at[pl.ds(off, rows)])

    @pl.when(sid == NS - 1)
    def _():
        rem_off = NS * rows
        pltpu.sync_copy(wx.at[pl.ds(rem_off, N_NODES - rem_off)],
                        shared.at[pl.ds(rem_off, N_NODES - rem_off)])

    pltpu.sync_copy(edst.at[pl.ds(base, EPW)], idx_v)
    plsc.subcore_barrier()
    _one_gather(shared, idx_v, out_i, base, rbuf, gsem, wsem)
    pltpu.sync_copy(esrc.at[pl.ds(base, EPW)], idx_v)
    _one_gather(shared, idx_v, out_j, base, rbuf, gsem, wsem)


def kernel(Wx, edge_index):
    eidx = edge_index.astype(jnp.int32)
    return _gather_both(Wx, eidx[0], eidx[1])


# speedup vs baseline: 11.0635x; 1.0659x over previous
import functools

import jax
import jax.numpy as jnp
from jax import lax
from jax.experimental import pallas as pl
from jax.experimental.pallas import tpu as pltpu
from jax.experimental.pallas import tpu_sc as plsc

N_NODES = 10000
N_EDGES = 320000
D = 128
NC, NS = 2, 16
NW = NC * NS
EPW = N_EDGES // NW
CH = 80
NFULL = EPW // CH
NB = 4

_mesh = plsc.VectorSubcoreMesh(core_axis_name="c", subcore_axis_name="s")


def _one_gather(wx, idx_v, out, base, rbuf, gsem, wsem):
    def g(c, b):
        pltpu.async_copy(wx.at[idx_v.at[pl.ds(c * CH, CH)]], rbuf.at[b], gsem)

    def gwait(b):
        pltpu.make_async_copy(
            wx.at[idx_v.at[pl.ds(0, CH)]], rbuf.at[b], gsem).wait()

    def w(c, b):
        pltpu.async_copy(rbuf.at[b], out.at[pl.ds(base + c * CH, CH)], wsem)

    def wwait():
        pltpu.make_async_copy(
            rbuf.at[0], out.at[pl.ds(base, CH)], wsem).wait()

    g(0, 0)
    g(1, 1)
    gwait(0); w(0, 0); g(2, 2)
    gwait(1); w(1, 1); g(3, 3)
    gwait(2); w(2, 2); wwait(); g(4, 0)
    gwait(3); w(3, 3); wwait(); g(5, 1)

    def outer(i, carry):
        c0 = 4 + i * 4
        for b in range(4):
            c = c0 + b
            bb = c % 4
            gwait(bb)
            w(c, bb)
            wwait()
            g(c + 2, (c + 2) % 4)
        return carry

    lax.fori_loop(0, (NFULL - 5 - 4) // 4, outer, 0)

    for c in (120, 121, 122):
        gwait(c % 4); w(c, c % 4); wwait(); g(c + 2, (c + 2) % 4)
    for c in (123, 124):
        gwait(c % 4); w(c, c % 4)
    for _ in range(4):
        wwait()


@functools.partial(
    pl.kernel,
    mesh=_mesh,
    out_type=(jax.ShapeDtypeStruct((N_EDGES, D), jnp.float32),
              jax.ShapeDtypeStruct((N_EDGES, D), jnp.float32)),
    scratch_types=[
        pltpu.VMEM((EPW,), jnp.int32),
        pltpu.VMEM((NB, CH, D), jnp.float32),
        pltpu.VMEM_SHARED((N_NODES, D), jnp.float32),
        pltpu.SemaphoreType.DMA,
        pltpu.SemaphoreType.DMA,
    ],
)
def _gather_both(wx, eidx, out_i, out_j, idx_v, rbuf, shared,
                 gsem, wsem):
    wid = lax.axis_index("s") * NC + lax.axis_index("c")
    base = wid * EPW
    sid = lax.axis_index("s")
    rows = (N_NODES // NS) // 8 * 8
    off = sid * rows
    pltpu.sync_copy(wx.at[pl.ds(off, rows)], shared.at[pl.ds(off, rows)])

    @pl.when(sid == NS - 1)
    def _():
        rem_off = NS * rows
        pltpu.sync_copy(wx.at[pl.ds(rem_off, N_NODES - rem_off)],
                        shared.at[pl.ds(rem_off, N_NODES - rem_off)])

    pltpu.sync_copy(eidx.at[pl.ds(N_EDGES + base, EPW)], idx_v)
    plsc.subcore_barrier()
    _one_gather(shared, idx_v, out_i, base, rbuf, gsem, wsem)
    pltpu.sync_copy(eidx.at[pl.ds(base, EPW)], idx_v)
    _one_gather(shared, idx_v, out_j, base, rbuf, gsem, wsem)


def kernel(Wx, edge_index):
    eidx = edge_index.astype(jnp.int32).reshape(-1)
    return _gather_both(Wx, eidx)


# speedup vs baseline: 11.0862x; 1.0020x over previous
import functools

import jax
import jax.numpy as jnp
from jax import lax
from jax.experimental import pallas as pl
from jax.experimental.pallas import tpu as pltpu
from jax.experimental.pallas import tpu_sc as plsc

N_NODES = 10000
N_EDGES = 320000
D = 128
NC_, NS = 2, 16
NW = NC_ * NS
EPW = N_EDGES // NW
CH = 40
NCHUNK = EPW // CH
NB = 8
LEAD = 4

assert EPW % CH == 0 and CH % 8 == 0 and LEAD <= NB - 2

_mesh = plsc.VectorSubcoreMesh(core_axis_name="c", subcore_axis_name="s")


def _one_gather(wx, idx_v, out, base, rbuf, gsem, wsem):
    def g(c, b):
        pltpu.async_copy(wx.at[idx_v.at[pl.ds(c * CH, CH)]], rbuf.at[b], gsem)

    def gwait(b):
        pltpu.make_async_copy(
            wx.at[idx_v.at[pl.ds(0, CH)]], rbuf.at[b], gsem).wait()

    def w(c, b):
        pltpu.async_copy(rbuf.at[b], out.at[pl.ds(base + c * CH, CH)], wsem)

    def wwait():
        pltpu.make_async_copy(
            rbuf.at[0], out.at[pl.ds(base, CH)], wsem).wait()

    def step(c, b):
        gwait(b)
        w(c, b)
        if c + LEAD < NCHUNK:
            if c + LEAD - NB >= 0:
                wwait()
            g(c + LEAD, (c + LEAD) % NB)

    def step_dyn(c, b):
        gwait(b)
        w(c, b)
        wwait()
        g(c + LEAD, (c + LEAD) % NB)

    for c in range(LEAD):
        g(c, c % NB)

    hi = NB + (NCHUNK - LEAD - NB) // NB * NB
    for c in range(NB):
        step(c, c % NB)

    def outer(i, carry):
        c0 = NB + i * NB
        for b in range(NB):
            step_dyn(c0 + b, b)
        return carry

    lax.fori_loop(0, (hi - NB) // NB, outer, 0)

    for c in range(hi, NCHUNK):
        step(c, c % NB)
    for _ in range(NB):
        wwait()


@functools.partial(
    pl.kernel,
    mesh=_mesh,
    out_type=(jax.ShapeDtypeStruct((N_EDGES, D), jnp.float32),
              jax.ShapeDtypeStruct((N_EDGES, D), jnp.float32)),
    scratch_types=[
        pltpu.VMEM((EPW,), jnp.int32),
        pltpu.VMEM((NB, CH, D), jnp.float32),
        pltpu.VMEM_SHARED((N_NODES, D), jnp.float32),
        pltpu.SemaphoreType.DMA,
        pltpu.SemaphoreType.DMA,
    ],
)
def _gather_both(wx, eidx, out_i, out_j, idx_v, rbuf, shared,
                 gsem, wsem):
    wid = lax.axis_index("s") * NC_ + lax.axis_index("c")
    base = wid * EPW
    sid = lax.axis_index("s")
    rows = (N_NODES // NS) // 8 * 8
    off = sid * rows
    pltpu.sync_copy(wx.at[pl.ds(off, rows)], shared.at[pl.ds(off, rows)])

    @pl.when(sid == NS - 1)
    def _():
        rem_off = NS * rows
        pltpu.sync_copy(wx.at[pl.ds(rem_off, N_NODES - rem_off)],
                        shared.at[pl.ds(rem_off, N_NODES - rem_off)])

    pltpu.sync_copy(eidx.at[pl.ds(N_EDGES + base, EPW)], idx_v)
    plsc.subcore_barrier()
    _one_gather(shared, idx_v, out_i, base, rbuf, gsem, wsem)
    pltpu.sync_copy(eidx.at[pl.ds(base, EPW)], idx_v)
    _one_gather(shared, idx_v, out_j, base, rbuf, gsem, wsem)


def kernel(Wx, edge_index):
    eidx = edge_index.astype(jnp.int32).reshape(-1)
    return _gather_both(Wx, eidx)
